# Initial kernel scaffold; baseline (speedup 1.0000x reference)
import jax
import jax.numpy as jnp
from jax.experimental import pallas as pl


def kernel(x, edge_index, edge_weight, W1, b1, W2, b2, g1, be1, g2, be2, Wih1, Whh1, bih1, bhh1, Wih2, Whh2, bih2, bhh2, linW, linb):
    raise NotImplementedError("write your pallas kernel here")


# speedup vs baseline: 12.3548x; 12.3548x over previous
import functools

import jax
import jax.numpy as jnp
from jax import lax
from jax.experimental import pallas as pl
from jax.experimental.pallas import tpu as pltpu
from jax.experimental.pallas import tpu_sc as plsc

N = 10000
E = 320000
F_IN = 128
H = 64
EPS = 1e-5

NC = 2
NS = 16
NW = NC * NS
CHUNK = 128
EPW = ((E + NW - 1) // NW + 2 * CHUNK - 1) // (2 * CHUNK) * (2 * CHUNK)
NCHUNK = EPW // CHUNK
E_PAD = EPW * NW
STRIPE = N // NS

_HI = lax.Precision.HIGHEST

def _sc_mesh():
    return plsc.VectorSubcoreMesh(core_axis_name="c", subcore_axis_name="s",
                                  num_cores=NC, num_subcores=NS)


def _deg_body(dst2d_hbm, ew_hbm, out_hbm, msg, ew_all, didx_all, didx, zrow, dacc):
    c = lax.axis_index("c")
    s = lax.axis_index("s")
    i32 = jnp.int32
    wid = c * i32(NS) + s
    zeros16 = jnp.zeros((16,), jnp.float32)

    pltpu.sync_copy(ew_hbm.at[pl.ds(wid * i32(EPW), EPW)], ew_all)
    pltpu.sync_copy(dst2d_hbm.at[pl.ds(wid * i32(NCHUNK), NCHUNK)], didx_all)

    def _z(i, carry):
        zrow[i, :] = zeros16
        return carry

    lax.fori_loop(i32(0), i32(STRIPE), _z, i32(0))
    pltpu.sync_copy(zrow, dacc.at[pl.ds(s * i32(STRIPE), STRIPE)])
    plsc.subcore_barrier()

    def _chunk(k, carry):
        ebase = k * i32(CHUNK)

        def _grp(g, cg):
            ewv = ew_all[pl.ds(ebase + g * i32(16), 16)]
            for j in range(16):
                msg[g * i32(16) + i32(j), :] = jnp.broadcast_to(ewv[j], (16,))
            return cg

        lax.fori_loop(i32(0), i32(CHUNK // 16), _grp, i32(0))
        for gg in range(CHUNK // 16):
            didx[pl.ds(gg * 16, 16)] = didx_all[k, pl.ds(gg * 16, 16)]
        pltpu.sync_copy(msg, dacc.at[didx], add=True)
        return carry

    lax.fori_loop(i32(0), i32(NCHUNK), _chunk, i32(0))
    plsc.subcore_barrier()
    pltpu.sync_copy(dacc.at[pl.ds(s * i32(STRIPE), STRIPE)],
                    out_hbm.at[c, pl.ds(s * i32(STRIPE), STRIPE)])


@functools.cache
def _deg_sc():
    return pl.kernel(
        _deg_body,
        out_type=jax.ShapeDtypeStruct((NC, N, 16), jnp.float32),
        mesh=_sc_mesh(),
        compiler_params=pltpu.CompilerParams(use_tc_tiling_on_sc=False),
        scratch_types=[
            pltpu.VMEM((CHUNK, 16), jnp.float32),
            pltpu.VMEM((EPW,), jnp.float32),
            pltpu.VMEM((NCHUNK, CHUNK), jnp.int32),
            pltpu.VMEM((CHUNK,), jnp.int32),
            pltpu.VMEM((STRIPE, 16), jnp.float32),
            pltpu.VMEM_SHARED((N, 16), jnp.float32),
        ],
    )


def _conv_body(xw_hbm, src2d_hbm, dst2d_hbm, ew_hbm, out_hbm,
               rows, sidx_all, didx_all, ew_all, sb0, sb1, didx,
               zbuf, acc, gsem):
    c = lax.axis_index("c")
    s = lax.axis_index("s")
    i32 = jnp.int32
    wid = c * i32(NS) + s
    zeros16 = jnp.zeros((16,), jnp.float32)

    pltpu.sync_copy(src2d_hbm.at[pl.ds(wid * i32(NCHUNK), NCHUNK)], sidx_all)
    pltpu.sync_copy(dst2d_hbm.at[pl.ds(wid * i32(NCHUNK), NCHUNK)], didx_all)
    pltpu.sync_copy(ew_hbm.at[pl.ds(wid * i32(EPW), EPW)], ew_all)

    def _z(i, carry):
        for q in range(H // 16):
            zbuf[i, pl.ds(q * 16, 16)] = zeros16
        return carry

    lax.fori_loop(i32(0), i32(STRIPE), _z, i32(0))
    pltpu.sync_copy(zbuf, acc.at[pl.ds(s * i32(STRIPE), STRIPE)])
    plsc.subcore_barrier()

    sbufs = (sb0, sb1)

    def _cpidx(dstbuf, k):
        for gg in range(CHUNK // 16):
            dstbuf[pl.ds(gg * 16, 16)] = sidx_all[k, pl.ds(gg * 16, 16)]

    _cpidx(sb0, i32(0))
    pltpu.async_copy(xw_hbm.at[sb0], rows.at[i32(0)], gsem)

    def _pair(kk, carry):
        for p in range(2):
            k = kk * i32(2) + i32(p)
            pltpu.make_async_copy(
                xw_hbm.at[sbufs[p]], rows.at[i32(p)], gsem).wait()

            @pl.when(k + i32(1) < i32(NCHUNK))
            def _():
                _cpidx(sbufs[1 - p], k + i32(1))
                pltpu.async_copy(
                    xw_hbm.at[sbufs[1 - p]], rows.at[i32(1 - p)], gsem)

            ebase = k * i32(CHUNK)

            def _grp(g, cg):
                ewv = ew_all[pl.ds(ebase + g * i32(16), 16)]
                for j in range(16):
                    sj = jnp.broadcast_to(ewv[j], (16,))
                    r = g * i32(16) + i32(j)
                    for q in range(H // 16):
                        rows[p, r, pl.ds(q * 16, 16)] = (
                            rows[p, r, pl.ds(q * 16, 16)] * sj)
                return cg

            lax.fori_loop(i32(0), i32(CHUNK // 16), _grp, i32(0))
            for gg in range(CHUNK // 16):
                didx[pl.ds(gg * 16, 16)] = didx_all[k, pl.ds(gg * 16, 16)]
            pltpu.sync_copy(rows.at[i32(p)], acc.at[didx], add=True)
        return carry

    lax.fori_loop(i32(0), i32(NCHUNK // 2), _pair, i32(0))
    plsc.subcore_barrier()
    pltpu.sync_copy(acc.at[pl.ds(s * i32(STRIPE), STRIPE)],
                    out_hbm.at[c, pl.ds(s * i32(STRIPE), STRIPE)])


@functools.cache
def _conv_sc():
    return pl.kernel(
        _conv_body,
        out_type=jax.ShapeDtypeStruct((NC, N, H), jnp.float32),
        mesh=_sc_mesh(),
        compiler_params=pltpu.CompilerParams(use_tc_tiling_on_sc=False),
        scratch_types=[
            pltpu.VMEM((2, CHUNK, H), jnp.float32),
            pltpu.VMEM((NCHUNK, CHUNK), jnp.int32),
            pltpu.VMEM((NCHUNK, CHUNK), jnp.int32),
            pltpu.VMEM((EPW,), jnp.float32),
            pltpu.VMEM((CHUNK,), jnp.int32),
            pltpu.VMEM((CHUNK,), jnp.int32),
            pltpu.VMEM((CHUNK,), jnp.int32),
            pltpu.VMEM((STRIPE, H), jnp.float32),
            pltpu.VMEM_SHARED((N, H), jnp.float32),
            pltpu.SemaphoreType.DMA,
        ],
    )


def _tca_body(degp, x, w1, dis_o, xw1s_o):
    d = degp[0][:, 0:1] + degp[1][:, 0:1] + 1.0
    dis = 1.0 / jnp.sqrt(d)
    dis_o[...] = dis
    xw = jnp.dot(x[...], w1[...], precision=_HI,
                 preferred_element_type=jnp.float32)
    xw1s_o[...] = xw * dis


def _bn_in(pre, g, be):
    z = jnp.maximum(pre, 0.0)
    m = jnp.mean(z, axis=0, keepdims=True)
    v = jnp.mean((z - m) ** 2, axis=0, keepdims=True)
    return (z - m) / jnp.sqrt(v + EPS) * g + be


def _tcb_body(accp, xw1s, dis, b1, g1, be1, w2, x1_o, xw2s_o):
    dis_v = dis[...]
    pre = dis_v * (accp[0] + accp[1] + xw1s[...]) + b1[...]
    x1 = _bn_in(pre, g1[...], be1[...])
    x1_o[...] = x1
    xw2s_o[...] = jnp.dot(x1, w2[...], precision=_HI,
                          preferred_element_type=jnp.float32) * dis_v


def _tcc1_body(accp, xw2s, dis, b2, g2, be2, x2_o):
    dis_v = dis[...]
    pre = dis_v * (accp[0] + accp[1] + xw2s[...]) + b2[...]
    x2_o[...] = _bn_in(pre, g2[...], be2[...])


def _tcc2_body(x1, x2, x,
               wih1t, bih1, bhh1, wih2t, bih2, bhh2, linw, linb, out_o):
    xc = jnp.concatenate([x1[...], x2[...]], axis=1)
    gates = jnp.dot(xc, wih1t[...], precision=_HI,
                    preferred_element_type=jnp.float32) + bih1[...] + bhh1[...]
    i1 = jax.nn.sigmoid(gates[:, 0:H])
    g1g = jnp.tanh(gates[:, 2 * H:3 * H])
    o1 = jax.nn.sigmoid(gates[:, 3 * H:4 * H])
    h1 = o1 * jnp.tanh(i1 * g1g)
    gates2 = jnp.dot(h1, wih2t[...], precision=_HI,
                     preferred_element_type=jnp.float32) + bih2[...] + bhh2[...]
    i2 = jax.nn.sigmoid(gates2[:, 0:H])
    g2g = jnp.tanh(gates2[:, 2 * H:3 * H])
    o2 = jax.nn.sigmoid(gates2[:, 3 * H:4 * H])
    h2 = o2 * jnp.tanh(i2 * g2g)
    hcat = jnp.concatenate([jnp.maximum(h1, 0.0), jnp.maximum(h2, 0.0),
                            jnp.maximum(x[...], 0.0)], axis=1)
    out_o[...] = jnp.dot(hcat, linw[...], precision=_HI,
                         preferred_element_type=jnp.float32) + linb[...]


def _tca(degp, x, w1):
    return pl.pallas_call(
        _tca_body,
        out_shape=[jax.ShapeDtypeStruct((N, 1), jnp.float32),
                   jax.ShapeDtypeStruct((N, H), jnp.float32)],
    )(degp, x, w1)


def _tcb(accp, xw1s, dis, b1, g1, be1, w2):
    return pl.pallas_call(
        _tcb_body,
        out_shape=[jax.ShapeDtypeStruct((N, H), jnp.float32),
                   jax.ShapeDtypeStruct((N, H), jnp.float32)],
    )(accp, xw1s, dis, b1, g1, be1, w2)


def _tcc1(accp, xw2s, dis, b2, g2, be2):
    return pl.pallas_call(
        _tcc1_body,
        out_shape=jax.ShapeDtypeStruct((N, H), jnp.float32),
    )(accp, xw2s, dis, b2, g2, be2)


_RB = 2000


def _tcc2(x1, x2, x, wih1t, bih1, bhh1, wih2t, bih2, bhh2, linw, linb):
    row = lambda w: pl.BlockSpec((_RB, w), lambda i: (i, i * 0))
    full = lambda a, b: pl.BlockSpec((a, b), lambda i: (i * 0, i * 0))
    return pl.pallas_call(
        _tcc2_body,
        grid=(N // _RB,),
        in_specs=[row(H), row(H), row(F_IN),
                  full(2 * H, 4 * H), full(1, 4 * H), full(1, 4 * H),
                  full(H, 4 * H), full(1, 4 * H), full(1, 4 * H),
                  full(2 * H + F_IN, 1), full(1, 1)],
        out_specs=row(1),
        out_shape=jax.ShapeDtypeStruct((N, 1), jnp.float32),
    )(x1, x2, x, wih1t, bih1, bhh1, wih2t, bih2, bhh2, linw, linb)


def kernel(x, edge_index, edge_weight, W1, b1, W2, b2, g1, be1, g2, be2,
           Wih1, Whh1, bih1, bhh1, Wih2, Whh2, bih2, bhh2, linW, linb):
    src = edge_index[0].astype(jnp.int32)
    dst = edge_index[1].astype(jnp.int32)
    ew = edge_weight.astype(jnp.float32)
    pad = E_PAD - E
    if pad:
        zi = jnp.zeros((pad,), jnp.int32)
        src = jnp.concatenate([src, zi])
        dst = jnp.concatenate([dst, zi])
        ew = jnp.concatenate([ew, jnp.zeros((pad,), jnp.float32)])
    x = x.astype(jnp.float32)

    src2d = src.reshape(NW * NCHUNK, CHUNK)
    dst2d = dst.reshape(NW * NCHUNK, CHUNK)

    degp = _deg_sc()(dst2d, ew)
    dis, xw1s = _tca(degp, x, W1.astype(jnp.float32))
    acc1 = _conv_sc()(xw1s, src2d, dst2d, ew)
    x1, xw2s = _tcb(acc1, xw1s, dis,
                    b1.reshape(1, H).astype(jnp.float32),
                    g1.reshape(1, H).astype(jnp.float32),
                    be1.reshape(1, H).astype(jnp.float32),
                    W2.astype(jnp.float32))
    acc2 = _conv_sc()(xw2s, src2d, dst2d, ew)
    x2 = _tcc1(acc2, xw2s, dis,
               b2.reshape(1, H).astype(jnp.float32),
               g2.reshape(1, H).astype(jnp.float32),
               be2.reshape(1, H).astype(jnp.float32))
    out = _tcc2(x1, x2, x,
                Wih1.astype(jnp.float32).T,
                bih1.reshape(1, 4 * H).astype(jnp.float32),
                bhh1.reshape(1, 4 * H).astype(jnp.float32),
                Wih2.astype(jnp.float32).T,
                bih2.reshape(1, 4 * H).astype(jnp.float32),
                bhh2.reshape(1, 4 * H).astype(jnp.float32),
                linW.astype(jnp.float32), linb.reshape(1, 1).astype(jnp.float32))
    return out
